# baseline (speedup 1.0000x reference)
import functools

import jax
import jax.numpy as jnp
from jax import lax
from jax.experimental import pallas as pl
from jax.experimental.pallas import tpu as pltpu
from jax.experimental.pallas import tpu_sc as plsc

CH = 256


@functools.lru_cache(maxsize=None)
def _build(b: int, l: int, d: int, vocab: int):
    info = plsc.get_sparse_core_info()
    nc, ns, nl = info.num_cores, info.num_subcores, info.num_lanes
    nw = nc * ns
    n_ch = b // CH
    n_slabs = l * n_ch
    assert b % CH == 0 and n_slabs % (2 * nw) == 0 and d % nl == 0
    slabs_pw = n_slabs // nw

    mesh = plsc.VectorSubcoreMesh(core_axis_name="c", subcore_axis_name="s")

    @functools.partial(
        pl.kernel,
        out_type=jax.ShapeDtypeStruct((l, d, b), jnp.float32),
        mesh=mesh,
        scratch_types=[
            pltpu.VMEM((CH,), jnp.int32),
            pltpu.VMEM((CH,), jnp.int32),
            pltpu.VMEM((CH, d), jnp.float32),
            pltpu.VMEM((CH, d), jnp.float32),
            pltpu.VMEM((d, CH), jnp.float32),
            pltpu.VMEM((d, CH), jnp.float32),
            pltpu.SemaphoreType.DMA,
            pltpu.SemaphoreType.DMA,
            pltpu.SemaphoreType.DMA,
            pltpu.SemaphoreType.DMA,
        ],
        compiler_params=pltpu.CompilerParams(use_tc_tiling_on_sc=False,
                                             needs_layout_passes=False),
    )
    def emb_kernel(x_hbm, emb_hbm, out_hbm, idx_a, idx_b, rows_a, rows_b,
                   t_a, t_b, gsem_a, gsem_b, osem_a, osem_b):
        wid = lax.axis_index("s") * nc + lax.axis_index("c")
        s0 = wid * slabs_pw
        iota = lax.iota(jnp.int32, nl)
        bufs = ((idx_a, rows_a, t_a, gsem_a, osem_a),
                (idx_b, rows_b, t_b, gsem_b, osem_b))

        def slab_lc(g):
            s = s0 + g
            return s // n_ch, lax.rem(s, n_ch)

        def issue(g, idx_v, rows, gsem):
            ll, ch = slab_lc(g)
            pltpu.sync_copy(x_hbm.at[ll, ch], idx_v)
            pltpu.make_async_copy(emb_hbm.at[idx_v], rows, gsem).start()

        def transpose(rows, tbuf):
            def c_body(c, carry):
                cvec = jnp.full((nl,), c, jnp.int32)
                for b0 in range(0, CH, nl):
                    col = plsc.load_gather(rows, [iota + b0, cvec])
                    tbuf[c, pl.ds(b0, nl)] = col
                return carry

            lax.fori_loop(0, d, c_body, 0)

        issue(0, idx_a, rows_a, gsem_a)
        issue(1, idx_b, rows_b, gsem_b)

        def pair_body(p, carry):
            g = 2 * p
            for parity, (idx_v, rows, tbuf, gsem, osem) in enumerate(bufs):
                gg = g + parity
                pltpu.make_async_copy(emb_hbm.at[idx_v], rows, gsem).wait()

                @pl.when(gg >= 2)
                def _():
                    ll, ch = slab_lc(gg - 2)
                    pltpu.make_async_copy(
                        tbuf, out_hbm.at[ll, :, pl.ds(ch * CH, CH)], osem,
                    ).wait()

                transpose(rows, tbuf)

                @pl.when(gg + 2 < slabs_pw)
                def _():
                    issue(gg + 2, idx_v, rows, gsem)

                ll, ch = slab_lc(gg)
                pltpu.make_async_copy(
                    tbuf, out_hbm.at[ll, :, pl.ds(ch * CH, CH)], osem,
                ).start()

            return carry

        lax.fori_loop(0, slabs_pw // 2, pair_body, 0)

        for parity, (idx_v, rows, tbuf, gsem, osem) in enumerate(bufs):
            ll, ch = slab_lc(slabs_pw - 2 + parity)
            pltpu.make_async_copy(
                tbuf, out_hbm.at[ll, :, pl.ds(ch * CH, CH)], osem,
            ).wait()

    return emb_kernel


def kernel(x, emb):
    b, l = x.shape
    d = emb.shape[1]
    xt = jnp.transpose(x.astype(jnp.int32)).reshape(l, b // CH, CH)
    out = _build(b, l, d, emb.shape[0])(xt, emb)
    return jnp.transpose(out, (2, 0, 1))


# speedup vs baseline: 1.6232x; 1.6232x over previous
import functools

import jax
import jax.numpy as jnp
from jax import lax
from jax.experimental import pallas as pl
from jax.experimental.pallas import tpu as pltpu
from jax.experimental.pallas import tpu_sc as plsc

CH = 256


@functools.lru_cache(maxsize=None)
def _build(b: int, l: int, d: int, vocab: int):
    info = plsc.get_sparse_core_info()
    nc, ns, nl = info.num_cores, info.num_subcores, info.num_lanes
    nw = nc * ns
    n_ch = b // CH
    n_slabs = l * n_ch
    assert b % CH == 0 and n_slabs % (2 * nw) == 0 and d % nl == 0
    slabs_pw = n_slabs // nw

    mesh = plsc.VectorSubcoreMesh(core_axis_name="c", subcore_axis_name="s")

    @functools.partial(
        pl.kernel,
        out_type=jax.ShapeDtypeStruct((l, d, b), jnp.float32),
        mesh=mesh,
        scratch_types=[
            pltpu.VMEM((CH,), jnp.int32),
            pltpu.VMEM((CH,), jnp.int32),
            pltpu.VMEM((CH, d), jnp.float32),
            pltpu.VMEM((CH, d), jnp.float32),
            pltpu.VMEM((d, CH), jnp.float32),
            pltpu.VMEM((d, CH), jnp.float32),
            pltpu.SemaphoreType.DMA,
            pltpu.SemaphoreType.DMA,
            pltpu.SemaphoreType.DMA,
            pltpu.SemaphoreType.DMA,
        ],
        compiler_params=pltpu.CompilerParams(use_tc_tiling_on_sc=False,
                                             needs_layout_passes=False),
    )
    def emb_kernel(x_hbm, emb_hbm, out_hbm, idx_a, idx_b, rows_a, rows_b,
                   t_a, t_b, gsem_a, gsem_b, osem_a, osem_b):
        wid = lax.axis_index("s") * nc + lax.axis_index("c")
        s0 = wid * slabs_pw
        iota = lax.iota(jnp.int32, nl)
        bufs = ((idx_a, rows_a, t_a, gsem_a, osem_a),
                (idx_b, rows_b, t_b, gsem_b, osem_b))

        def slab_lc(g):
            s = s0 + g
            return s // n_ch, lax.rem(s, n_ch)

        def issue(g, idx_v, rows, gsem):
            ll, ch = slab_lc(g)
            pltpu.sync_copy(x_hbm.at[ll, ch], idx_v)
            pltpu.make_async_copy(emb_hbm.at[idx_v], rows, gsem).start()

        rots = [jnp.bitwise_and(iota + j, nl - 1) for j in range(nl)]
        n_blk_b = CH // nl

        def transpose(rows, tbuf):
            def blk_body(bi, carry):
                c0 = (bi // n_blk_b) * nl
                b0 = lax.rem(bi, n_blk_b) * nl
                bidx = iota + b0
                for j in range(nl):
                    cidx = rots[j] + c0
                    vals = plsc.load_gather(rows, [bidx, cidx])
                    plsc.store_scatter(tbuf, [cidx, bidx], vals)
                return carry

            lax.fori_loop(0, n_blk_b * (d // nl), blk_body, 0)

        issue(0, idx_a, rows_a, gsem_a)
        issue(1, idx_b, rows_b, gsem_b)

        def pair_body(p, carry):
            g = 2 * p
            for parity, (idx_v, rows, tbuf, gsem, osem) in enumerate(bufs):
                gg = g + parity
                pltpu.make_async_copy(emb_hbm.at[idx_v], rows, gsem).wait()

                @pl.when(gg >= 2)
                def _():
                    ll, ch = slab_lc(gg - 2)
                    pltpu.make_async_copy(
                        tbuf, out_hbm.at[ll, :, pl.ds(ch * CH, CH)], osem,
                    ).wait()

                transpose(rows, tbuf)

                @pl.when(gg + 2 < slabs_pw)
                def _():
                    issue(gg + 2, idx_v, rows, gsem)

                ll, ch = slab_lc(gg)
                pltpu.make_async_copy(
                    tbuf, out_hbm.at[ll, :, pl.ds(ch * CH, CH)], osem,
                ).start()

            return carry

        lax.fori_loop(0, slabs_pw // 2, pair_body, 0)

        for parity, (idx_v, rows, tbuf, gsem, osem) in enumerate(bufs):
            ll, ch = slab_lc(slabs_pw - 2 + parity)
            pltpu.make_async_copy(
                tbuf, out_hbm.at[ll, :, pl.ds(ch * CH, CH)], osem,
            ).wait()

    return emb_kernel


def kernel(x, emb):
    b, l = x.shape
    d = emb.shape[1]
    xt = jnp.transpose(x.astype(jnp.int32)).reshape(l, b // CH, CH)
    out = _build(b, l, d, emb.shape[0])(xt, emb)
    return jnp.transpose(out, (2, 0, 1))


# speedup vs baseline: 1.7409x; 1.0725x over previous
import functools

import jax
import jax.numpy as jnp
from jax import lax
from jax.experimental import pallas as pl
from jax.experimental.pallas import tpu as pltpu
from jax.experimental.pallas import tpu_sc as plsc

CH = 256


@functools.lru_cache(maxsize=None)
def _build(b: int, l: int, d: int, vocab: int):
    info = plsc.get_sparse_core_info()
    nc, ns, nl = info.num_cores, info.num_subcores, info.num_lanes
    nw = nc * ns
    n_ch = b // CH
    n_slabs = l * n_ch
    assert b % CH == 0 and n_slabs % (2 * nw) == 0 and d % nl == 0
    slabs_pw = n_slabs // nw

    mesh = plsc.VectorSubcoreMesh(core_axis_name="c", subcore_axis_name="s")

    @functools.partial(
        pl.kernel,
        out_type=jax.ShapeDtypeStruct((l, d, b), jnp.float32),
        mesh=mesh,
        scratch_types=[
            pltpu.VMEM((CH,), jnp.int32),
            pltpu.VMEM((CH,), jnp.int32),
            pltpu.VMEM((CH, d), jnp.float32),
            pltpu.VMEM((CH, d), jnp.float32),
            pltpu.VMEM((d, CH), jnp.float32),
            pltpu.VMEM((d, CH), jnp.float32),
            pltpu.SemaphoreType.DMA,
            pltpu.SemaphoreType.DMA,
            pltpu.SemaphoreType.DMA,
            pltpu.SemaphoreType.DMA,
        ],
        compiler_params=pltpu.CompilerParams(use_tc_tiling_on_sc=False,
                                             needs_layout_passes=False),
    )
    def emb_kernel(x_hbm, emb_hbm, out_hbm, idx_a, idx_b, rows_a, rows_b,
                   t_a, t_b, gsem_a, gsem_b, osem_a, osem_b):
        wid = lax.axis_index("s") * nc + lax.axis_index("c")
        s0 = wid * slabs_pw
        iota = lax.iota(jnp.int32, nl)
        bufs = ((idx_a, rows_a, t_a, gsem_a, osem_a),
                (idx_b, rows_b, t_b, gsem_b, osem_b))

        def slab_lc(g):
            s = s0 + g
            return s // n_ch, lax.rem(s, n_ch)

        def issue(g, idx_v, rows, gsem):
            ll, ch = slab_lc(g)
            pltpu.sync_copy(x_hbm.at[ll, ch], idx_v)
            pltpu.make_async_copy(emb_hbm.at[idx_v], rows, gsem).start()

        rots = [jnp.bitwise_and(iota + j, nl - 1) for j in range(nl)]
        n_blk_b = CH // nl

        def transpose(rows, tbuf):
            @plsc.parallel_loop(0, n_blk_b, step=1, unroll=2)
            def _(bi):
                b0 = bi * nl
                bidx = iota + b0
                for cb in range(d // nl):
                    for j in range(nl):
                        cidx = rots[j] + cb * nl
                        vals = plsc.load_gather(rows, [bidx, cidx])
                        plsc.store_scatter(tbuf, [cidx, bidx], vals)

        issue(0, idx_a, rows_a, gsem_a)
        issue(1, idx_b, rows_b, gsem_b)

        def pair_body(p, carry):
            g = 2 * p
            for parity, (idx_v, rows, tbuf, gsem, osem) in enumerate(bufs):
                gg = g + parity
                pltpu.make_async_copy(emb_hbm.at[idx_v], rows, gsem).wait()

                @pl.when(gg >= 2)
                def _():
                    ll, ch = slab_lc(gg - 2)
                    pltpu.make_async_copy(
                        tbuf, out_hbm.at[ll, :, pl.ds(ch * CH, CH)], osem,
                    ).wait()

                transpose(rows, tbuf)

                @pl.when(gg + 2 < slabs_pw)
                def _():
                    issue(gg + 2, idx_v, rows, gsem)

                ll, ch = slab_lc(gg)
                pltpu.make_async_copy(
                    tbuf, out_hbm.at[ll, :, pl.ds(ch * CH, CH)], osem,
                ).start()

            return carry

        lax.fori_loop(0, slabs_pw // 2, pair_body, 0)

        for parity, (idx_v, rows, tbuf, gsem, osem) in enumerate(bufs):
            ll, ch = slab_lc(slabs_pw - 2 + parity)
            pltpu.make_async_copy(
                tbuf, out_hbm.at[ll, :, pl.ds(ch * CH, CH)], osem,
            ).wait()

    return emb_kernel


def kernel(x, emb):
    b, l = x.shape
    d = emb.shape[1]
    xt = jnp.transpose(x.astype(jnp.int32)).reshape(l, b // CH, CH)
    out = _build(b, l, d, emb.shape[0])(xt, emb)
    return jnp.transpose(out, (2, 0, 1))


# speedup vs baseline: 2.0386x; 1.1710x over previous
import functools

import jax
import jax.numpy as jnp
from jax import lax
from jax.experimental import pallas as pl
from jax.experimental.pallas import tpu as pltpu
from jax.experimental.pallas import tpu_sc as plsc

CH = 256


@functools.lru_cache(maxsize=None)
def _build(b: int, l: int, d: int, vocab: int):
    info = plsc.get_sparse_core_info()
    nc, ns, nl = info.num_cores, info.num_subcores, info.num_lanes
    nw = nc * ns
    n_ch = b // CH
    n_slabs = l * n_ch
    assert b % CH == 0 and n_slabs % (2 * nw) == 0 and d % nl == 0
    slabs_pw = n_slabs // nw

    mesh = plsc.VectorSubcoreMesh(core_axis_name="c", subcore_axis_name="s")

    @functools.partial(
        pl.kernel,
        out_type=jax.ShapeDtypeStruct((l, d, b), jnp.float32),
        mesh=mesh,
        scratch_types=[
            pltpu.VMEM((CH,), jnp.int32),
            pltpu.VMEM((CH,), jnp.int32),
            pltpu.VMEM((CH, d), jnp.float32),
            pltpu.VMEM((CH, d), jnp.float32),
            pltpu.VMEM((d, CH), jnp.float32),
            pltpu.VMEM((d, CH), jnp.float32),
            pltpu.SemaphoreType.DMA,
            pltpu.SemaphoreType.DMA,
            pltpu.SemaphoreType.DMA,
            pltpu.SemaphoreType.DMA,
        ],
        compiler_params=pltpu.CompilerParams(use_tc_tiling_on_sc=False,
                                             needs_layout_passes=False),
    )
    def emb_kernel(x_hbm, emb_hbm, out_hbm, idx_a, idx_b, rows_a, rows_b,
                   t_a, t_b, gsem_a, gsem_b, osem_a, osem_b):
        wid = lax.axis_index("s") * nc + lax.axis_index("c")
        s0 = wid * slabs_pw
        iota = lax.iota(jnp.int32, nl)
        bufs = ((idx_a, rows_a, t_a, gsem_a, osem_a),
                (idx_b, rows_b, t_b, gsem_b, osem_b))

        def slab_lc(g):
            s = s0 + g
            return s // n_ch, lax.rem(s, n_ch)

        def issue(g, idx_v, rows, gsem):
            ll, ch = slab_lc(g)
            pltpu.sync_copy(x_hbm.at[ll, ch], idx_v)
            pltpu.make_async_copy(emb_hbm.at[idx_v], rows, gsem).start()

        rots = [jnp.bitwise_and(iota + j, nl - 1) for j in range(nl)]
        n_blk_b = CH // nl

        def transpose(rows, tbuf):
            @plsc.parallel_loop(0, n_blk_b, step=1, unroll=4)
            def _(bi):
                b0 = bi * nl
                bidx = iota + b0
                for cb in range(d // nl):
                    for j in range(nl):
                        cidx = rots[j] + cb * nl
                        vals = plsc.load_gather(rows, [bidx, cidx])
                        plsc.store_scatter(tbuf, [cidx, bidx], vals)

        issue(0, idx_a, rows_a, gsem_a)
        issue(1, idx_b, rows_b, gsem_b)

        def pair_body(p, carry):
            g = 2 * p
            for parity, (idx_v, rows, tbuf, gsem, osem) in enumerate(bufs):
                gg = g + parity
                pltpu.make_async_copy(emb_hbm.at[idx_v], rows, gsem).wait()

                @pl.when(gg >= 2)
                def _():
                    ll, ch = slab_lc(gg - 2)
                    pltpu.make_async_copy(
                        tbuf, out_hbm.at[ll, :, pl.ds(ch * CH, CH)], osem,
                    ).wait()

                transpose(rows, tbuf)

                @pl.when(gg + 2 < slabs_pw)
                def _():
                    issue(gg + 2, idx_v, rows, gsem)

                ll, ch = slab_lc(gg)
                pltpu.make_async_copy(
                    tbuf, out_hbm.at[ll, :, pl.ds(ch * CH, CH)], osem,
                ).start()

            return carry

        lax.fori_loop(0, slabs_pw // 2, pair_body, 0)

        for parity, (idx_v, rows, tbuf, gsem, osem) in enumerate(bufs):
            ll, ch = slab_lc(slabs_pw - 2 + parity)
            pltpu.make_async_copy(
                tbuf, out_hbm.at[ll, :, pl.ds(ch * CH, CH)], osem,
            ).wait()

    return emb_kernel


def kernel(x, emb):
    b, l = x.shape
    d = emb.shape[1]
    xt = jnp.transpose(x.astype(jnp.int32)).reshape(l, b // CH, CH)
    out = _build(b, l, d, emb.shape[0])(xt, emb)
    return jnp.transpose(out, (2, 0, 1))
